# Initial kernel scaffold; baseline (speedup 1.0000x reference)
import jax
import jax.numpy as jnp
from jax.experimental import pallas as pl


def kernel(x, weight):
    raise NotImplementedError("write your pallas kernel here")


# speedup vs baseline: 1.1013x; 1.1013x over previous
import functools

import jax
import jax.numpy as jnp
from jax import lax
from jax.experimental import pallas as pl
from jax.experimental.pallas import tpu as pltpu
from jax.experimental.pallas import tpu_sc as plsc

DIM = 64
B = 4096
FIELDS = 26
TOTAL = B * FIELDS
CHUNK = 128
NC = 2
NS = 16
NW = NC * NS
CPW = TOTAL // (NW * CHUNK)

_mesh = plsc.VectorSubcoreMesh(core_axis_name="c", subcore_axis_name="s")


@functools.partial(
    pl.kernel,
    mesh=_mesh,
    out_type=jax.ShapeDtypeStruct((TOTAL, DIM), jnp.float32),
    scratch_types=[
        pltpu.VMEM((CPW, CHUNK), jnp.int32),
        pltpu.VMEM((CHUNK, DIM), jnp.float32),
        pltpu.SemaphoreType.DMA,
    ],
    compiler_params=pltpu.CompilerParams(use_tc_tiling_on_sc=False),
)
def _sc_gather(x_hbm, w_hbm, out_hbm, idx_v, buf, sem):
    wid = lax.axis_index("s") * NC + lax.axis_index("c")
    base = wid * CPW
    pltpu.sync_copy(x_hbm.at[wid], idx_v)

    def body(j, _):
        pltpu.async_copy(w_hbm.at[idx_v.at[j]], buf, sem).wait()
        off = pl.multiple_of((base + j) * CHUNK, CHUNK)
        pltpu.sync_copy(buf, out_hbm.at[pl.ds(off, CHUNK)])
        return 0

    lax.fori_loop(0, CPW, body, 0)


def kernel(x, weight):
    xr = x.reshape(NW, CPW, CHUNK)
    out = _sc_gather(xr, weight)
    return out.reshape(B, FIELDS, DIM)


# speedup vs baseline: 1.2090x; 1.0978x over previous
import functools

import jax
import jax.numpy as jnp
from jax import lax
from jax.experimental import pallas as pl
from jax.experimental.pallas import tpu as pltpu
from jax.experimental.pallas import tpu_sc as plsc

DIM = 64
B = 4096
FIELDS = 26
TOTAL = B * FIELDS
CHUNK = 128
NC = 2
NS = 16
NW = NC * NS
CPW = TOTAL // (NW * CHUNK)
NBUF = 6
GLAG = 3

_mesh = plsc.VectorSubcoreMesh(core_axis_name="c", subcore_axis_name="s")


@functools.partial(
    pl.kernel,
    mesh=_mesh,
    out_type=jax.ShapeDtypeStruct((TOTAL, DIM), jnp.float32),
    scratch_types=(
        [pltpu.VMEM((CPW, CHUNK), jnp.int32)]
        + [pltpu.VMEM((CHUNK, DIM), jnp.float32) for _ in range(NBUF)]
        + [pltpu.SemaphoreType.DMA for _ in range(2 * NBUF)]
    ),
    compiler_params=pltpu.CompilerParams(use_tc_tiling_on_sc=False),
)
def _sc_gather(x_hbm, w_hbm, out_hbm, idx_v, *rest):
    bufs = rest[:NBUF]
    sg = rest[NBUF:2 * NBUF]
    ss = rest[2 * NBUF:3 * NBUF]
    wid = lax.axis_index("s") * NC + lax.axis_index("c")
    base = wid * CPW
    pltpu.sync_copy(x_hbm.at[wid], idx_v)

    gd = [None] * CPW
    sd = [None] * CPW

    def start_gather(j):
        b = j % NBUF
        gd[j] = pltpu.async_copy(w_hbm.at[idx_v.at[j]], bufs[b], sg[b])

    def start_store(j):
        b = j % NBUF
        off = pl.multiple_of((base + j) * CHUNK, CHUNK)
        sd[j] = pltpu.async_copy(bufs[b], out_hbm.at[pl.ds(off, CHUNK)], ss[b])

    for j in range(GLAG):
        start_gather(j)
    for j in range(CPW):
        gd[j].wait()
        start_store(j)
        nj = j + GLAG
        if nj < CPW:
            pj = nj - NBUF
            if pj >= 0:
                sd[pj].wait()
            start_gather(nj)
    for j in range(CPW - NBUF, CPW):
        sd[j].wait()


def kernel(x, weight):
    xr = x.reshape(NW, CPW, CHUNK)
    out = _sc_gather(xr, weight)
    return out.reshape(B, FIELDS, DIM)
